# baseline (speedup 1.0000x reference)
import functools

import jax
import jax.numpy as jnp
from jax.experimental import pallas as pl
from jax.experimental.pallas import tpu as pltpu

_ALPHA = 0.1
_EPS = 1e-12


def _body(x_ref, p_ref, soft_ref, hard_ref, pn_ref):
    @pl.when(pl.program_id(0) == 0)
    def _():
        p = p_ref[...]
        n = jnp.sqrt(jnp.sum(p * p, axis=-1, keepdims=True))
        pn_ref[...] = p / jnp.maximum(n, _EPS)

    x = x_ref[...]
    xn = x / jnp.maximum(jnp.sqrt(jnp.sum(x * x, axis=-1, keepdims=True)), _EPS)
    sim = jax.lax.dot_general(
        xn, pn_ref[...],
        dimension_numbers=(((1,), (1,)), ((), ())),
        preferred_element_type=jnp.float32,
    )
    e = jnp.exp(_ALPHA * sim)
    ones = jnp.ones((sim.shape[-1], 8), dtype=jnp.float32)
    s = jax.lax.dot_general(
        e, ones,
        dimension_numbers=(((1,), (0,)), ((), ())),
        preferred_element_type=jnp.float32,
    )[:, 0:1]
    soft_ref[...] = e * (1.0 / s)
    hard_ref[...] = jnp.argmax(sim, axis=-1, keepdims=True).astype(jnp.int32)


@jax.jit
def kernel(x, prototypes):
    B, N, C = x.shape
    K = prototypes.shape[0]
    R = B * N
    x_flat = x.reshape(R, C)
    TR = 512
    grid = (R // TR,)
    soft, hard = pl.pallas_call(
        _body,
        grid=grid,
        in_specs=[
            pl.BlockSpec((TR, C), lambda i: (i, 0)),
            pl.BlockSpec((K, C), lambda i: (0, 0)),
        ],
        out_specs=[
            pl.BlockSpec((TR, K), lambda i: (i, 0)),
            pl.BlockSpec((TR, 1), lambda i: (i, 0)),
        ],
        out_shape=[
            jax.ShapeDtypeStruct((R, K), jnp.float32),
            jax.ShapeDtypeStruct((R, 1), jnp.int32),
        ],
        scratch_shapes=[pltpu.VMEM((K, C), jnp.float32)],
        compiler_params=pltpu.CompilerParams(
            dimension_semantics=("arbitrary",),
        ),
    )(x_flat, prototypes)
    return soft.reshape(B, N, K), hard.reshape(B, N)
